# Initial kernel scaffold; baseline (speedup 1.0000x reference)
import jax
import jax.numpy as jnp
from jax.experimental import pallas as pl


def kernel(x, W_qkv, b_qkv, W_o, b_o, lepe_w, lepe_b):
    raise NotImplementedError("write your pallas kernel here")


# speedup vs baseline: 1.6717x; 1.6717x over previous
import functools

import jax
import jax.numpy as jnp
from jax.experimental import pallas as pl
from jax.experimental.pallas import tpu as pltpu

DIM = 256
QK = 256
HEADS = 8
TOPK = 2
WV = 2
SCALE = QK ** -0.5
CH = QK // HEADS
CV = DIM // HEADS
W2 = 128
NWIN = 32


def _qkv_kernel(x_ref, w_ref, b_ref, qkv_ref, qwin_ref, kwin_ref):
    acc = jnp.dot(x_ref[...], w_ref[...], preferred_element_type=jnp.float32)
    acc = acc + b_ref[...]
    qkv_ref[...] = acc
    qwin_ref[...] = jnp.mean(acc[:, :QK], axis=0, keepdims=True)[None]
    kwin_ref[...] = jnp.mean(acc[:, QK:2 * QK], axis=0, keepdims=True)[None]


def _router_kernel(qwin_ref, kwin_ref, idx_ref):
    qw = qwin_ref[...].reshape(-1, QK)
    kw = kwin_ref[...].reshape(-1, QK)
    for n in range(2):
        q = qw[n * NWIN:(n + 1) * NWIN, :] * SCALE
        k = kw[n * NWIN:(n + 1) * NWIN, :]
        logits = jnp.dot(q, k.T, preferred_element_type=jnp.float32)
        cols = jax.lax.broadcasted_iota(jnp.int32, logits.shape, 1)
        big = jnp.int32(NWIN)
        m0 = jnp.max(logits, axis=-1, keepdims=True)
        i0 = jnp.min(jnp.where(logits == m0, cols, big), axis=-1,
                     keepdims=True)
        masked = jnp.where(cols == i0, -jnp.inf, logits)
        m1 = jnp.max(masked, axis=-1, keepdims=True)
        i1 = jnp.min(jnp.where(masked == m1, cols, big), axis=-1,
                     keepdims=True)
        base = jnp.int32(n * NWIN)
        idx_ref[n * NWIN:(n + 1) * NWIN, :] = (
            jnp.concatenate([i0, i1], axis=-1) + base)


def _lepe_kernel(v_ref, w_ref, out_ref):
    v = v_ref[...].reshape(64, 64, DIM)
    vp = jnp.pad(v, ((1, 1), (1, 1), (0, 0)))
    acc = jnp.zeros((64, 64, DIM), jnp.float32)
    for dy in range(3):
        for dx in range(3):
            acc = acc + vp[dy:dy + 64, dx:dx + 64, :] * w_ref[3 * dy + dx, :]
    out_ref[...] = acc.reshape(64 * 64, DIM)


def _attn_kernel(idx_ref, q_ref, ka_ref, kb_ref, va_ref, vb_ref, lepe_ref,
                 wo_ref, bo_ref, out_ref):
    del idx_ref
    q = q_ref[...] * SCALE
    k = jnp.concatenate([ka_ref[...], kb_ref[...]], axis=0)
    v = jnp.concatenate([va_ref[...], vb_ref[...]], axis=0)
    outs = []
    for h in range(HEADS):
        sl = slice(h * CH, (h + 1) * CH)
        logits = jnp.dot(q[:, sl], k[:, sl].T,
                         preferred_element_type=jnp.float32)
        p = jax.nn.softmax(logits, axis=-1)
        outs.append(jnp.dot(p, v[:, sl], preferred_element_type=jnp.float32))
    attn = jnp.concatenate(outs, axis=-1)
    acc = attn + lepe_ref[...]
    out_ref[...] = (jnp.dot(acc, wo_ref[...], preferred_element_type=jnp.float32)
                    + bo_ref[...])


@functools.partial(jax.jit, static_argnames=())
def _forward_impl(x, W_qkv, b_qkv, W_o, b_o, lepe_w, lepe_b):
    N, C, H, W = x.shape
    P = N * H * W
    nwin_total = N * NWIN
    xp = jnp.transpose(x, (0, 2, 3, 1)).reshape(P, C)
    b2 = b_qkv.reshape(1, -1)

    qkv, qwin, kwin = pl.pallas_call(
        _qkv_kernel,
        grid=(nwin_total,),
        in_specs=[
            pl.BlockSpec((W2, C), lambda i: (i, 0)),
            pl.BlockSpec((C, 2 * QK + DIM), lambda i: (0, 0)),
            pl.BlockSpec((1, 2 * QK + DIM), lambda i: (0, 0)),
        ],
        out_specs=[
            pl.BlockSpec((W2, 2 * QK + DIM), lambda i: (i, 0)),
            pl.BlockSpec((1, 1, QK), lambda i: (i, 0, 0)),
            pl.BlockSpec((1, 1, QK), lambda i: (i, 0, 0)),
        ],
        out_shape=[
            jax.ShapeDtypeStruct((P, 2 * QK + DIM), jnp.float32),
            jax.ShapeDtypeStruct((nwin_total, 1, QK), jnp.float32),
            jax.ShapeDtypeStruct((nwin_total, 1, QK), jnp.float32),
        ],
    )(xp, W_qkv, b2)

    idx2d = pl.pallas_call(
        _router_kernel,
        grid=(1,),
        in_specs=[
            pl.BlockSpec((nwin_total, 1, QK), lambda i: (0, 0, 0)),
            pl.BlockSpec((nwin_total, 1, QK), lambda i: (0, 0, 0)),
        ],
        out_specs=pl.BlockSpec((nwin_total, TOPK), lambda i: (0, 0)),
        out_shape=jax.ShapeDtypeStruct((nwin_total, TOPK), jnp.int32),
    )(qwin, kwin)

    lw = jnp.transpose(lepe_w.reshape(DIM, 3, 3), (1, 2, 0)).reshape(9, DIM)
    lepe = pl.pallas_call(
        _lepe_kernel,
        grid=(N,),
        in_specs=[
            pl.BlockSpec((H * W, DIM), lambda n: (n, 2)),
            pl.BlockSpec((9, DIM), lambda n: (0, 0)),
        ],
        out_specs=pl.BlockSpec((H * W, DIM), lambda n: (n, 0)),
        out_shape=jax.ShapeDtypeStruct((P, DIM), jnp.float32),
    )(qkv, lw)
    b_eff = (b_o + lepe_b @ W_o).reshape(1, -1)

    grid_spec = pltpu.PrefetchScalarGridSpec(
        num_scalar_prefetch=1,
        grid=(nwin_total,),
        in_specs=[
            pl.BlockSpec((W2, QK), lambda i, s: (i, 0)),
            pl.BlockSpec((W2, QK), lambda i, s: (s[i, 0], 1)),
            pl.BlockSpec((W2, QK), lambda i, s: (s[i, 1], 1)),
            pl.BlockSpec((W2, DIM), lambda i, s: (s[i, 0], 2)),
            pl.BlockSpec((W2, DIM), lambda i, s: (s[i, 1], 2)),
            pl.BlockSpec((W2, DIM), lambda i, s: (i, 0)),
            pl.BlockSpec((DIM, DIM), lambda i, s: (0, 0)),
            pl.BlockSpec((1, DIM), lambda i, s: (0, 0)),
        ],
        out_specs=pl.BlockSpec((W2, DIM), lambda i, s: (i, 0)),
    )
    out = pl.pallas_call(
        _attn_kernel,
        grid_spec=grid_spec,
        out_shape=jax.ShapeDtypeStruct((P, DIM), jnp.float32),
    )(idx2d, qkv, qkv, qkv, qkv, qkv, lepe, W_o, b_eff)

    return out.reshape(N, H, W, DIM)


def kernel(x, W_qkv, b_qkv, W_o, b_o, lepe_w, lepe_b):
    return _forward_impl(x, W_qkv, b_qkv, W_o, b_o, lepe_w, lepe_b)


# speedup vs baseline: 1.7959x; 1.0743x over previous
import functools

import jax
import jax.numpy as jnp
from jax.experimental import pallas as pl
from jax.experimental.pallas import tpu as pltpu

DIM = 256
QK = 256
HEADS = 8
TOPK = 2
WV = 2
SCALE = QK ** -0.5
CH = QK // HEADS
CV = DIM // HEADS
W2 = 128
NWIN = 32


def _qkv_kernel(x_ref, w_ref, b_ref, qkv_ref, qwin_ref, kwin_ref):
    acc = jnp.dot(x_ref[...], w_ref[...], preferred_element_type=jnp.float32)
    acc = acc + b_ref[...]
    qkv_ref[...] = acc
    qwin_ref[...] = jnp.mean(acc[:, :QK], axis=0, keepdims=True)[None]
    kwin_ref[...] = jnp.mean(acc[:, QK:2 * QK], axis=0, keepdims=True)[None]


def _router_kernel(qwin_ref, kwin_ref, idx_ref):
    qw = qwin_ref[...].reshape(-1, QK)
    kw = kwin_ref[...].reshape(-1, QK)
    for n in range(2):
        q = qw[n * NWIN:(n + 1) * NWIN, :] * SCALE
        k = kw[n * NWIN:(n + 1) * NWIN, :]
        logits = jnp.dot(q, k.T, preferred_element_type=jnp.float32)
        cols = jax.lax.broadcasted_iota(jnp.int32, logits.shape, 1)
        big = jnp.int32(NWIN)
        m0 = jnp.max(logits, axis=-1, keepdims=True)
        i0 = jnp.min(jnp.where(logits == m0, cols, big), axis=-1,
                     keepdims=True)
        masked = jnp.where(cols == i0, -jnp.inf, logits)
        m1 = jnp.max(masked, axis=-1, keepdims=True)
        i1 = jnp.min(jnp.where(masked == m1, cols, big), axis=-1,
                     keepdims=True)
        base = jnp.int32(n * NWIN)
        idx_ref[n * NWIN:(n + 1) * NWIN, :] = (
            jnp.concatenate([i0, i1], axis=-1) + base)


def _lepe_kernel(v_ref, w_ref, out_ref):
    v = v_ref[...].reshape(64, 64, DIM)
    zr = jnp.zeros((1, 64, DIM), jnp.float32)
    zc = jnp.zeros((64, 1, DIM), jnp.float32)

    def shift(a, dy, dx):
        if dy == -1:
            a = jnp.concatenate([zr, a[:-1]], axis=0)
        elif dy == 1:
            a = jnp.concatenate([a[1:], zr], axis=0)
        if dx == -1:
            a = jnp.concatenate([zc, a[:, :-1]], axis=1)
        elif dx == 1:
            a = jnp.concatenate([a[:, 1:], zc], axis=1)
        return a

    acc = v * w_ref[4, :]
    for dy in (-1, 0, 1):
        for dx in (-1, 0, 1):
            if dy == 0 and dx == 0:
                continue
            acc = acc + shift(v, dy, dx) * w_ref[3 * (dy + 1) + (dx + 1), :]
    out_ref[...] = acc.reshape(64 * 64, DIM)


def _attn_kernel(idx_ref, q_ref, ka0_ref, kb0_ref, ka1_ref, kb1_ref,
                 va0_ref, vb0_ref, va1_ref, vb1_ref, lepe_ref,
                 wo_ref, bo_ref, out_ref):
    del idx_ref
    q_all = q_ref[...] * SCALE
    for w, (ka, kb, va, vb) in enumerate(
            ((ka0_ref, kb0_ref, va0_ref, vb0_ref),
             (ka1_ref, kb1_ref, va1_ref, vb1_ref))):
        q = q_all[w * W2:(w + 1) * W2, :]
        k = jnp.concatenate([ka[...], kb[...]], axis=0)
        v = jnp.concatenate([va[...], vb[...]], axis=0)
        kt = k.T
        outs = []
        for h in range(HEADS):
            sl = slice(h * CH, (h + 1) * CH)
            logits = jnp.dot(q[:, sl], kt[sl, :],
                             preferred_element_type=jnp.float32)
            p = jax.nn.softmax(logits, axis=-1)
            outs.append(jnp.dot(p, v[:, sl],
                                preferred_element_type=jnp.float32))
        attn = jnp.concatenate(outs, axis=-1)
        acc = attn + lepe_ref[w * W2:(w + 1) * W2, :]
        out_ref[w * W2:(w + 1) * W2, :] = (
            jnp.dot(acc, wo_ref[...], preferred_element_type=jnp.float32)
            + bo_ref[...])


@functools.partial(jax.jit, static_argnames=())
def _forward_impl(x, W_qkv, b_qkv, W_o, b_o, lepe_w, lepe_b):
    N, C, H, W = x.shape
    P = N * H * W
    nwin_total = N * NWIN
    xp = jnp.transpose(x, (0, 2, 3, 1)).reshape(P, C)
    b2 = b_qkv.reshape(1, -1)

    qkv, qwin, kwin = pl.pallas_call(
        _qkv_kernel,
        grid=(nwin_total,),
        in_specs=[
            pl.BlockSpec((W2, C), lambda i: (i, 0)),
            pl.BlockSpec((C, 2 * QK + DIM), lambda i: (0, 0)),
            pl.BlockSpec((1, 2 * QK + DIM), lambda i: (0, 0)),
        ],
        out_specs=[
            pl.BlockSpec((W2, 2 * QK + DIM), lambda i: (i, 0)),
            pl.BlockSpec((1, 1, QK), lambda i: (i, 0, 0)),
            pl.BlockSpec((1, 1, QK), lambda i: (i, 0, 0)),
        ],
        out_shape=[
            jax.ShapeDtypeStruct((P, 2 * QK + DIM), jnp.float32),
            jax.ShapeDtypeStruct((nwin_total, 1, QK), jnp.float32),
            jax.ShapeDtypeStruct((nwin_total, 1, QK), jnp.float32),
        ],
    )(xp, W_qkv, b2)

    idx2d = pl.pallas_call(
        _router_kernel,
        grid=(1,),
        in_specs=[
            pl.BlockSpec((nwin_total, 1, QK), lambda i: (0, 0, 0)),
            pl.BlockSpec((nwin_total, 1, QK), lambda i: (0, 0, 0)),
        ],
        out_specs=pl.BlockSpec((nwin_total, TOPK), lambda i: (0, 0)),
        out_shape=jax.ShapeDtypeStruct((nwin_total, TOPK), jnp.int32),
    )(qwin, kwin)

    lw = jnp.transpose(lepe_w.reshape(DIM, 3, 3), (1, 2, 0)).reshape(9, DIM)
    lepe = pl.pallas_call(
        _lepe_kernel,
        grid=(N,),
        in_specs=[
            pl.BlockSpec((H * W, DIM), lambda n: (n, 2)),
            pl.BlockSpec((9, DIM), lambda n: (0, 0)),
        ],
        out_specs=pl.BlockSpec((H * W, DIM), lambda n: (n, 0)),
        out_shape=jax.ShapeDtypeStruct((P, DIM), jnp.float32),
    )(qkv, lw)
    b_eff = (b_o + lepe_b @ W_o).reshape(1, -1)

    grid_spec = pltpu.PrefetchScalarGridSpec(
        num_scalar_prefetch=1,
        grid=(nwin_total // 2,),
        in_specs=[
            pl.BlockSpec((2 * W2, QK), lambda i, s: (i, 0)),
            pl.BlockSpec((W2, QK), lambda i, s: (s[2 * i, 0], 1)),
            pl.BlockSpec((W2, QK), lambda i, s: (s[2 * i, 1], 1)),
            pl.BlockSpec((W2, QK), lambda i, s: (s[2 * i + 1, 0], 1)),
            pl.BlockSpec((W2, QK), lambda i, s: (s[2 * i + 1, 1], 1)),
            pl.BlockSpec((W2, DIM), lambda i, s: (s[2 * i, 0], 2)),
            pl.BlockSpec((W2, DIM), lambda i, s: (s[2 * i, 1], 2)),
            pl.BlockSpec((W2, DIM), lambda i, s: (s[2 * i + 1, 0], 2)),
            pl.BlockSpec((W2, DIM), lambda i, s: (s[2 * i + 1, 1], 2)),
            pl.BlockSpec((2 * W2, DIM), lambda i, s: (i, 0)),
            pl.BlockSpec((DIM, DIM), lambda i, s: (0, 0)),
            pl.BlockSpec((1, DIM), lambda i, s: (0, 0)),
        ],
        out_specs=pl.BlockSpec((2 * W2, DIM), lambda i, s: (i, 0)),
    )
    out = pl.pallas_call(
        _attn_kernel,
        grid_spec=grid_spec,
        out_shape=jax.ShapeDtypeStruct((P, DIM), jnp.float32),
    )(idx2d, qkv, qkv, qkv, qkv, qkv, qkv, qkv, qkv, qkv, lepe, W_o, b_eff)

    return out.reshape(N, H, W, DIM)


def kernel(x, W_qkv, b_qkv, W_o, b_o, lepe_w, lepe_b):
    return _forward_impl(x, W_qkv, b_qkv, W_o, b_o, lepe_w, lepe_b)


# speedup vs baseline: 2.1308x; 1.1864x over previous
import functools

import jax
import jax.numpy as jnp
from jax.experimental import pallas as pl
from jax.experimental.pallas import tpu as pltpu

DIM = 256
QK = 256
HEADS = 8
TOPK = 2
WV = 2
SCALE = QK ** -0.5
CH = QK // HEADS
CV = DIM // HEADS
W2 = 128
NWIN = 32


def _qkv_kernel(x_ref, w_ref, b_ref, qkv_ref, qwin_ref, kwin_ref):
    acc = jnp.dot(x_ref[...], w_ref[...], preferred_element_type=jnp.float32)
    acc = acc + b_ref[...]
    qkv_ref[...] = acc
    qwin_ref[...] = jnp.mean(acc[:, :QK], axis=0, keepdims=True)[None]
    kwin_ref[...] = jnp.mean(acc[:, QK:2 * QK], axis=0, keepdims=True)[None]


def _router_kernel(qwin_ref, kwin_ref, idx_ref):
    qw = qwin_ref[...].reshape(-1, QK)
    kw = kwin_ref[...].reshape(-1, QK)
    for n in range(2):
        q = qw[n * NWIN:(n + 1) * NWIN, :] * SCALE
        k = kw[n * NWIN:(n + 1) * NWIN, :]
        logits = jnp.dot(q, k.T, preferred_element_type=jnp.float32)
        cols = jax.lax.broadcasted_iota(jnp.int32, logits.shape, 1)
        big = jnp.int32(NWIN)
        m0 = jnp.max(logits, axis=-1, keepdims=True)
        i0 = jnp.min(jnp.where(logits == m0, cols, big), axis=-1,
                     keepdims=True)
        masked = jnp.where(cols == i0, -jnp.inf, logits)
        m1 = jnp.max(masked, axis=-1, keepdims=True)
        i1 = jnp.min(jnp.where(masked == m1, cols, big), axis=-1,
                     keepdims=True)
        base = jnp.int32(n * NWIN)
        idx_ref[n * NWIN:(n + 1) * NWIN, :] = (
            jnp.concatenate([i0, i1], axis=-1) + base)


def _lepe_kernel(v_ref, w_ref, out_ref):
    v = v_ref[...].reshape(64, 64, DIM)
    zr = jnp.zeros((1, 64, DIM), jnp.float32)
    zc = jnp.zeros((64, 1, DIM), jnp.float32)

    def shift(a, dy, dx):
        if dy == -1:
            a = jnp.concatenate([zr, a[:-1]], axis=0)
        elif dy == 1:
            a = jnp.concatenate([a[1:], zr], axis=0)
        if dx == -1:
            a = jnp.concatenate([zc, a[:, :-1]], axis=1)
        elif dx == 1:
            a = jnp.concatenate([a[:, 1:], zc], axis=1)
        return a

    acc = v * w_ref[4, :]
    for dy in (-1, 0, 1):
        for dx in (-1, 0, 1):
            if dy == 0 and dx == 0:
                continue
            acc = acc + shift(v, dy, dx) * w_ref[3 * (dy + 1) + (dx + 1), :]
    out_ref[...] = acc.reshape(64 * 64, DIM)


def _attn_kernel(idx_ref, q_ref, ka0_ref, kb0_ref, ka1_ref, kb1_ref,
                 va0_ref, vb0_ref, va1_ref, vb1_ref, lepe_ref,
                 wo_ref, bo_ref, out_ref):
    del idx_ref
    q_all = q_ref[...] * SCALE
    for w, (ka, kb, va, vb) in enumerate(
            ((ka0_ref, kb0_ref, va0_ref, vb0_ref),
             (ka1_ref, kb1_ref, va1_ref, vb1_ref))):
        q = q_all[w * W2:(w + 1) * W2, :]
        k = jnp.concatenate([ka[...], kb[...]], axis=0)
        v = jnp.concatenate([va[...], vb[...]], axis=0)
        kt = k.T
        outs = []
        for h in range(HEADS):
            sl = slice(h * CH, (h + 1) * CH)
            logits = jnp.dot(q[:, sl], kt[sl, :],
                             preferred_element_type=jnp.float32)
            e = jnp.exp(logits)
            r = 1.0 / jnp.sum(e, axis=-1, keepdims=True)
            outs.append(jnp.dot(e, v[:, sl],
                                preferred_element_type=jnp.float32) * r)
        attn = jnp.concatenate(outs, axis=-1)
        acc = attn + lepe_ref[w * W2:(w + 1) * W2, :]
        out_ref[w * W2:(w + 1) * W2, :] = (
            jnp.dot(acc, wo_ref[...], preferred_element_type=jnp.float32)
            + bo_ref[...])


@functools.partial(jax.jit, static_argnames=())
def _forward_impl(x, W_qkv, b_qkv, W_o, b_o, lepe_w, lepe_b):
    N, C, H, W = x.shape
    P = N * H * W
    nwin_total = N * NWIN
    xp = jnp.transpose(x, (0, 2, 3, 1)).reshape(P, C)
    b2 = b_qkv.reshape(1, -1)

    qkv, qwin, kwin = pl.pallas_call(
        _qkv_kernel,
        grid=(nwin_total,),
        in_specs=[
            pl.BlockSpec((W2, C), lambda i: (i, 0)),
            pl.BlockSpec((C, 2 * QK + DIM), lambda i: (0, 0)),
            pl.BlockSpec((1, 2 * QK + DIM), lambda i: (0, 0)),
        ],
        out_specs=[
            pl.BlockSpec((W2, 2 * QK + DIM), lambda i: (i, 0)),
            pl.BlockSpec((1, 1, QK), lambda i: (i, 0, 0)),
            pl.BlockSpec((1, 1, QK), lambda i: (i, 0, 0)),
        ],
        out_shape=[
            jax.ShapeDtypeStruct((P, 2 * QK + DIM), jnp.float32),
            jax.ShapeDtypeStruct((nwin_total, 1, QK), jnp.float32),
            jax.ShapeDtypeStruct((nwin_total, 1, QK), jnp.float32),
        ],
    )(xp, W_qkv, b2)

    idx2d = pl.pallas_call(
        _router_kernel,
        grid=(1,),
        in_specs=[
            pl.BlockSpec((nwin_total, 1, QK), lambda i: (0, 0, 0)),
            pl.BlockSpec((nwin_total, 1, QK), lambda i: (0, 0, 0)),
        ],
        out_specs=pl.BlockSpec((nwin_total, TOPK), lambda i: (0, 0)),
        out_shape=jax.ShapeDtypeStruct((nwin_total, TOPK), jnp.int32),
    )(qwin, kwin)

    lw = jnp.transpose(lepe_w.reshape(DIM, 3, 3), (1, 2, 0)).reshape(9, DIM)
    lepe = pl.pallas_call(
        _lepe_kernel,
        grid=(N,),
        in_specs=[
            pl.BlockSpec((H * W, DIM), lambda n: (n, 2)),
            pl.BlockSpec((9, DIM), lambda n: (0, 0)),
        ],
        out_specs=pl.BlockSpec((H * W, DIM), lambda n: (n, 0)),
        out_shape=jax.ShapeDtypeStruct((P, DIM), jnp.float32),
    )(qkv, lw)
    b_eff = (b_o + lepe_b @ W_o).reshape(1, -1)

    grid_spec = pltpu.PrefetchScalarGridSpec(
        num_scalar_prefetch=1,
        grid=(nwin_total // 2,),
        in_specs=[
            pl.BlockSpec((2 * W2, QK), lambda i, s: (i, 0)),
            pl.BlockSpec((W2, QK), lambda i, s: (s[2 * i, 0], 1)),
            pl.BlockSpec((W2, QK), lambda i, s: (s[2 * i, 1], 1)),
            pl.BlockSpec((W2, QK), lambda i, s: (s[2 * i + 1, 0], 1)),
            pl.BlockSpec((W2, QK), lambda i, s: (s[2 * i + 1, 1], 1)),
            pl.BlockSpec((W2, DIM), lambda i, s: (s[2 * i, 0], 2)),
            pl.BlockSpec((W2, DIM), lambda i, s: (s[2 * i, 1], 2)),
            pl.BlockSpec((W2, DIM), lambda i, s: (s[2 * i + 1, 0], 2)),
            pl.BlockSpec((W2, DIM), lambda i, s: (s[2 * i + 1, 1], 2)),
            pl.BlockSpec((2 * W2, DIM), lambda i, s: (i, 0)),
            pl.BlockSpec((DIM, DIM), lambda i, s: (0, 0)),
            pl.BlockSpec((1, DIM), lambda i, s: (0, 0)),
        ],
        out_specs=pl.BlockSpec((2 * W2, DIM), lambda i, s: (i, 0)),
    )
    out = pl.pallas_call(
        _attn_kernel,
        grid_spec=grid_spec,
        out_shape=jax.ShapeDtypeStruct((P, DIM), jnp.float32),
    )(idx2d, qkv, qkv, qkv, qkv, qkv, qkv, qkv, qkv, qkv, lepe, W_o, b_eff)

    return out.reshape(N, H, W, DIM)


def kernel(x, W_qkv, b_qkv, W_o, b_o, lepe_w, lepe_b):
    return _forward_impl(x, W_qkv, b_qkv, W_o, b_o, lepe_w, lepe_b)


# speedup vs baseline: 2.1681x; 1.0175x over previous
import functools

import jax
import jax.numpy as jnp
from jax.experimental import pallas as pl
from jax.experimental.pallas import tpu as pltpu

DIM = 256
QK = 256
HEADS = 8
TOPK = 2
WV = 2
SCALE = QK ** -0.5
CH = QK // HEADS
CV = DIM // HEADS
W2 = 128
NWIN = 32


def _qkv_kernel(x_ref, w_ref, b_ref, qkv_ref, qwin_ref, kwin_ref):
    acc = jnp.dot(x_ref[...], w_ref[...], preferred_element_type=jnp.float32)
    acc = acc + b_ref[...]
    qkv_ref[...] = acc
    for w in range(acc.shape[0] // W2):
        blk = acc[w * W2:(w + 1) * W2, :]
        qwin_ref[w, :, :] = jnp.mean(blk[:, :QK], axis=0, keepdims=True)
        kwin_ref[w, :, :] = jnp.mean(blk[:, QK:2 * QK], axis=0, keepdims=True)


def _router_kernel(qwin_ref, kwin_ref, idx_ref):
    qw = qwin_ref[...].reshape(-1, QK)
    kw = kwin_ref[...].reshape(-1, QK)
    for n in range(2):
        q = qw[n * NWIN:(n + 1) * NWIN, :] * SCALE
        k = kw[n * NWIN:(n + 1) * NWIN, :]
        logits = jnp.dot(q, k.T, preferred_element_type=jnp.float32)
        cols = jax.lax.broadcasted_iota(jnp.int32, logits.shape, 1)
        big = jnp.int32(NWIN)
        m0 = jnp.max(logits, axis=-1, keepdims=True)
        i0 = jnp.min(jnp.where(logits == m0, cols, big), axis=-1,
                     keepdims=True)
        masked = jnp.where(cols == i0, -jnp.inf, logits)
        m1 = jnp.max(masked, axis=-1, keepdims=True)
        i1 = jnp.min(jnp.where(masked == m1, cols, big), axis=-1,
                     keepdims=True)
        base = jnp.int32(n * NWIN)
        idx_ref[n * NWIN:(n + 1) * NWIN, :] = (
            jnp.concatenate([i0, i1], axis=-1) + base)


def _lepe_kernel(v_ref, w_ref, out_ref):
    v = v_ref[...].reshape(64, 64, DIM)
    zr = jnp.zeros((1, 64, DIM), jnp.float32)
    zc = jnp.zeros((64, 1, DIM), jnp.float32)

    def shift(a, dy, dx):
        if dy == -1:
            a = jnp.concatenate([zr, a[:-1]], axis=0)
        elif dy == 1:
            a = jnp.concatenate([a[1:], zr], axis=0)
        if dx == -1:
            a = jnp.concatenate([zc, a[:, :-1]], axis=1)
        elif dx == 1:
            a = jnp.concatenate([a[:, 1:], zc], axis=1)
        return a

    acc = v * w_ref[4, :]
    for dy in (-1, 0, 1):
        for dx in (-1, 0, 1):
            if dy == 0 and dx == 0:
                continue
            acc = acc + shift(v, dy, dx) * w_ref[3 * (dy + 1) + (dx + 1), :]
    out_ref[...] = acc.reshape(64 * 64, DIM)


WPS = 4


def _attn_kernel(idx_ref, q_ref, *refs):
    del idx_ref
    gather = refs[:4 * WPS]
    lepe_ref, wo_ref, bo_ref, out_ref = refs[4 * WPS:]
    q_all = q_ref[...] * SCALE
    for w in range(WPS):
        ka, kb, va, vb = gather[4 * w:4 * w + 4]
        q = q_all[w * W2:(w + 1) * W2, :]
        k = jnp.concatenate([ka[...], kb[...]], axis=0)
        v = jnp.concatenate([va[...], vb[...]], axis=0)
        kt = k.T
        outs = []
        for h in range(HEADS):
            sl = slice(h * CH, (h + 1) * CH)
            logits = jnp.dot(q[:, sl], kt[sl, :],
                             preferred_element_type=jnp.float32)
            e = jnp.exp(logits)
            r = 1.0 / jnp.sum(e, axis=-1, keepdims=True)
            outs.append(jnp.dot(e, v[:, sl],
                                preferred_element_type=jnp.float32) * r)
        attn = jnp.concatenate(outs, axis=-1)
        acc = attn + lepe_ref[w * W2:(w + 1) * W2, :]
        out_ref[w * W2:(w + 1) * W2, :] = (
            jnp.dot(acc, wo_ref[...], preferred_element_type=jnp.float32)
            + bo_ref[...])


@functools.partial(jax.jit, static_argnames=())
def _forward_impl(x, W_qkv, b_qkv, W_o, b_o, lepe_w, lepe_b):
    N, C, H, W = x.shape
    P = N * H * W
    nwin_total = N * NWIN
    xp = jnp.transpose(x, (0, 2, 3, 1)).reshape(P, C)
    b2 = b_qkv.reshape(1, -1)

    QT = 4
    qkv, qwin, kwin = pl.pallas_call(
        _qkv_kernel,
        grid=(nwin_total // QT,),
        in_specs=[
            pl.BlockSpec((QT * W2, C), lambda i: (i, 0)),
            pl.BlockSpec((C, 2 * QK + DIM), lambda i: (0, 0)),
            pl.BlockSpec((1, 2 * QK + DIM), lambda i: (0, 0)),
        ],
        out_specs=[
            pl.BlockSpec((QT * W2, 2 * QK + DIM), lambda i: (i, 0)),
            pl.BlockSpec((QT, 1, QK), lambda i: (i, 0, 0)),
            pl.BlockSpec((QT, 1, QK), lambda i: (i, 0, 0)),
        ],
        out_shape=[
            jax.ShapeDtypeStruct((P, 2 * QK + DIM), jnp.float32),
            jax.ShapeDtypeStruct((nwin_total, 1, QK), jnp.float32),
            jax.ShapeDtypeStruct((nwin_total, 1, QK), jnp.float32),
        ],
    )(xp, W_qkv, b2)

    idx2d = pl.pallas_call(
        _router_kernel,
        grid=(1,),
        in_specs=[
            pl.BlockSpec((nwin_total, 1, QK), lambda i: (0, 0, 0)),
            pl.BlockSpec((nwin_total, 1, QK), lambda i: (0, 0, 0)),
        ],
        out_specs=pl.BlockSpec((nwin_total, TOPK), lambda i: (0, 0)),
        out_shape=jax.ShapeDtypeStruct((nwin_total, TOPK), jnp.int32),
    )(qwin, kwin)

    lw = jnp.transpose(lepe_w.reshape(DIM, 3, 3), (1, 2, 0)).reshape(9, DIM)
    lepe = pl.pallas_call(
        _lepe_kernel,
        grid=(N,),
        in_specs=[
            pl.BlockSpec((H * W, DIM), lambda n: (n, 2)),
            pl.BlockSpec((9, DIM), lambda n: (0, 0)),
        ],
        out_specs=pl.BlockSpec((H * W, DIM), lambda n: (n, 0)),
        out_shape=jax.ShapeDtypeStruct((P, DIM), jnp.float32),
    )(qkv, lw)
    b_eff = (b_o + lepe_b @ W_o).reshape(1, -1)

    gather_specs = []
    for w in range(WPS):
        gather_specs += [
            pl.BlockSpec((W2, QK), lambda i, s, w=w: (s[WPS * i + w, 0], 1)),
            pl.BlockSpec((W2, QK), lambda i, s, w=w: (s[WPS * i + w, 1], 1)),
            pl.BlockSpec((W2, DIM), lambda i, s, w=w: (s[WPS * i + w, 0], 2)),
            pl.BlockSpec((W2, DIM), lambda i, s, w=w: (s[WPS * i + w, 1], 2)),
        ]
    grid_spec = pltpu.PrefetchScalarGridSpec(
        num_scalar_prefetch=1,
        grid=(nwin_total // WPS,),
        in_specs=(
            [pl.BlockSpec((WPS * W2, QK), lambda i, s: (i, 0))]
            + gather_specs
            + [pl.BlockSpec((WPS * W2, DIM), lambda i, s: (i, 0)),
               pl.BlockSpec((DIM, DIM), lambda i, s: (0, 0)),
               pl.BlockSpec((1, DIM), lambda i, s: (0, 0))]
        ),
        out_specs=pl.BlockSpec((WPS * W2, DIM), lambda i, s: (i, 0)),
    )
    out = pl.pallas_call(
        _attn_kernel,
        grid_spec=grid_spec,
        out_shape=jax.ShapeDtypeStruct((P, DIM), jnp.float32),
    )(idx2d, qkv, *([qkv] * (4 * WPS)), lepe, W_o, b_eff)

    return out.reshape(N, H, W, DIM)


def kernel(x, W_qkv, b_qkv, W_o, b_o, lepe_w, lepe_b):
    return _forward_impl(x, W_qkv, b_qkv, W_o, b_o, lepe_w, lepe_b)


# speedup vs baseline: 2.2957x; 1.0589x over previous
import functools

import jax
import jax.numpy as jnp
from jax.experimental import pallas as pl
from jax.experimental.pallas import tpu as pltpu

DIM = 256
QK = 256
HEADS = 8
TOPK = 2
WV = 2
SCALE = QK ** -0.5
CH = QK // HEADS
CV = DIM // HEADS
W2 = 128
NWIN = 32


def _qkv_kernel(x_ref, w_ref, b_ref, qkv_ref, qwin_ref, kwin_ref):
    acc = jnp.dot(x_ref[...], w_ref[...], preferred_element_type=jnp.float32)
    acc = acc + b_ref[...]
    qkv_ref[...] = acc
    for w in range(acc.shape[0] // W2):
        blk = acc[w * W2:(w + 1) * W2, :]
        qwin_ref[w, :, :] = jnp.mean(blk[:, :QK], axis=0, keepdims=True)
        kwin_ref[w, :, :] = jnp.mean(blk[:, QK:2 * QK], axis=0, keepdims=True)


def _router_kernel(qwin_ref, kwin_ref, idx_ref):
    qw = qwin_ref[...].reshape(-1, QK)
    kw = kwin_ref[...].reshape(-1, QK)
    for n in range(2):
        q = qw[n * NWIN:(n + 1) * NWIN, :] * SCALE
        k = kw[n * NWIN:(n + 1) * NWIN, :]
        logits = jnp.dot(q, k.T, preferred_element_type=jnp.float32)
        cols = jax.lax.broadcasted_iota(jnp.int32, logits.shape, 1)
        big = jnp.int32(NWIN)
        m0 = jnp.max(logits, axis=-1, keepdims=True)
        i0 = jnp.min(jnp.where(logits == m0, cols, big), axis=-1,
                     keepdims=True)
        masked = jnp.where(cols == i0, -jnp.inf, logits)
        m1 = jnp.max(masked, axis=-1, keepdims=True)
        i1 = jnp.min(jnp.where(masked == m1, cols, big), axis=-1,
                     keepdims=True)
        base = jnp.int32(n * NWIN)
        idx_ref[n * NWIN:(n + 1) * NWIN, :] = (
            jnp.concatenate([i0, i1], axis=-1) + base)


def _lepe_kernel(v_ref, w_ref, out_ref):
    v = v_ref[...].reshape(64, 64, DIM)
    zr = jnp.zeros((1, 64, DIM), jnp.float32)
    zc = jnp.zeros((64, 1, DIM), jnp.float32)

    def shift(a, dy, dx):
        if dy == -1:
            a = jnp.concatenate([zr, a[:-1]], axis=0)
        elif dy == 1:
            a = jnp.concatenate([a[1:], zr], axis=0)
        if dx == -1:
            a = jnp.concatenate([zc, a[:, :-1]], axis=1)
        elif dx == 1:
            a = jnp.concatenate([a[:, 1:], zc], axis=1)
        return a

    acc = v * w_ref[4, :]
    for dy in (-1, 0, 1):
        for dx in (-1, 0, 1):
            if dy == 0 and dx == 0:
                continue
            acc = acc + shift(v, dy, dx) * w_ref[3 * (dy + 1) + (dx + 1), :]
    out_ref[...] = acc.reshape(64 * 64, DIM)


WPS = 4


def _attn_kernel(idx_ref, q_ref, *refs):
    del idx_ref
    gather = refs[:4 * WPS]
    lepe_ref, wo_ref, bo_ref, out_ref = refs[4 * WPS:]
    for w in range(WPS):
        ka, kb, va, vb = gather[4 * w:4 * w + 4]
        q = q_ref[w * W2:(w + 1) * W2, :] * SCALE
        kta = ka[...].T
        ktb = kb[...].T
        va_ = va[...]
        vb_ = vb[...]
        outs = []
        for h in range(HEADS):
            sl = slice(h * CH, (h + 1) * CH)
            ea = jnp.exp(jnp.dot(q[:, sl], kta[sl, :],
                                 preferred_element_type=jnp.float32))
            eb = jnp.exp(jnp.dot(q[:, sl], ktb[sl, :],
                                 preferred_element_type=jnp.float32))
            s = (jnp.sum(ea, axis=-1, keepdims=True)
                 + jnp.sum(eb, axis=-1, keepdims=True))
            pv = (jnp.dot(ea, va_[:, sl], preferred_element_type=jnp.float32)
                  + jnp.dot(eb, vb_[:, sl], preferred_element_type=jnp.float32))
            outs.append(pv * (1.0 / s))
        attn = jnp.concatenate(outs, axis=-1)
        acc = attn + lepe_ref[w * W2:(w + 1) * W2, :]
        out_ref[w * W2:(w + 1) * W2, :] = (
            jnp.dot(acc, wo_ref[...], preferred_element_type=jnp.float32)
            + bo_ref[...])


@functools.partial(jax.jit, static_argnames=())
def _forward_impl(x, W_qkv, b_qkv, W_o, b_o, lepe_w, lepe_b):
    N, C, H, W = x.shape
    P = N * H * W
    nwin_total = N * NWIN
    xp = jnp.transpose(x, (0, 2, 3, 1)).reshape(P, C)
    b2 = b_qkv.reshape(1, -1)

    QT = 4
    qkv, qwin, kwin = pl.pallas_call(
        _qkv_kernel,
        grid=(nwin_total // QT,),
        in_specs=[
            pl.BlockSpec((QT * W2, C), lambda i: (i, 0)),
            pl.BlockSpec((C, 2 * QK + DIM), lambda i: (0, 0)),
            pl.BlockSpec((1, 2 * QK + DIM), lambda i: (0, 0)),
        ],
        out_specs=[
            pl.BlockSpec((QT * W2, 2 * QK + DIM), lambda i: (i, 0)),
            pl.BlockSpec((QT, 1, QK), lambda i: (i, 0, 0)),
            pl.BlockSpec((QT, 1, QK), lambda i: (i, 0, 0)),
        ],
        out_shape=[
            jax.ShapeDtypeStruct((P, 2 * QK + DIM), jnp.float32),
            jax.ShapeDtypeStruct((nwin_total, 1, QK), jnp.float32),
            jax.ShapeDtypeStruct((nwin_total, 1, QK), jnp.float32),
        ],
    )(xp, W_qkv, b2)

    idx2d = pl.pallas_call(
        _router_kernel,
        grid=(1,),
        in_specs=[
            pl.BlockSpec((nwin_total, 1, QK), lambda i: (0, 0, 0)),
            pl.BlockSpec((nwin_total, 1, QK), lambda i: (0, 0, 0)),
        ],
        out_specs=pl.BlockSpec((nwin_total, TOPK), lambda i: (0, 0)),
        out_shape=jax.ShapeDtypeStruct((nwin_total, TOPK), jnp.int32),
    )(qwin, kwin)

    lw = jnp.transpose(lepe_w.reshape(DIM, 3, 3), (1, 2, 0)).reshape(9, DIM)
    lepe = pl.pallas_call(
        _lepe_kernel,
        grid=(N,),
        in_specs=[
            pl.BlockSpec((H * W, DIM), lambda n: (n, 2)),
            pl.BlockSpec((9, DIM), lambda n: (0, 0)),
        ],
        out_specs=pl.BlockSpec((H * W, DIM), lambda n: (n, 0)),
        out_shape=jax.ShapeDtypeStruct((P, DIM), jnp.float32),
    )(qkv, lw)
    b_eff = (b_o + lepe_b @ W_o).reshape(1, -1)

    gather_specs = []
    for w in range(WPS):
        gather_specs += [
            pl.BlockSpec((W2, QK), lambda i, s, w=w: (s[WPS * i + w, 0], 1)),
            pl.BlockSpec((W2, QK), lambda i, s, w=w: (s[WPS * i + w, 1], 1)),
            pl.BlockSpec((W2, DIM), lambda i, s, w=w: (s[WPS * i + w, 0], 2)),
            pl.BlockSpec((W2, DIM), lambda i, s, w=w: (s[WPS * i + w, 1], 2)),
        ]
    grid_spec = pltpu.PrefetchScalarGridSpec(
        num_scalar_prefetch=1,
        grid=(nwin_total // WPS,),
        in_specs=(
            [pl.BlockSpec((WPS * W2, QK), lambda i, s: (i, 0))]
            + gather_specs
            + [pl.BlockSpec((WPS * W2, DIM), lambda i, s: (i, 0)),
               pl.BlockSpec((DIM, DIM), lambda i, s: (0, 0)),
               pl.BlockSpec((1, DIM), lambda i, s: (0, 0))]
        ),
        out_specs=pl.BlockSpec((WPS * W2, DIM), lambda i, s: (i, 0)),
    )
    out = pl.pallas_call(
        _attn_kernel,
        grid_spec=grid_spec,
        out_shape=jax.ShapeDtypeStruct((P, DIM), jnp.float32),
    )(idx2d, qkv, *([qkv] * (4 * WPS)), lepe, W_o, b_eff)

    return out.reshape(N, H, W, DIM)


def kernel(x, W_qkv, b_qkv, W_o, b_o, lepe_w, lepe_b):
    return _forward_impl(x, W_qkv, b_qkv, W_o, b_o, lepe_w, lepe_b)
